# baseline (speedup 1.0000x reference)
import jax
import jax.numpy as jnp
from jax.experimental import pallas as pl
from jax.experimental.pallas import tpu as pltpu

_BM = 400


def _linear_body(x_ref, w_ref, b_ref, o_ref):
    o_ref[...] = (
        jnp.dot(x_ref[...], w_ref[...], preferred_element_type=jnp.float32)
        + b_ref[...]
    )


def _layer1_body(g_ref, a_ref, w2_ref, b2_ref, o_ref):
    h = jnp.dot(g_ref[...], a_ref[...], preferred_element_type=jnp.float32)
    h = jnp.maximum(h, 0.0)
    o_ref[...] = (
        jnp.dot(h, w2_ref[...], preferred_element_type=jnp.float32) + b2_ref[...]
    )


def _layer2_body(g_ref, b_ref, o_ref):
    o_ref[...] = jnp.dot(g_ref[...], b_ref[...], preferred_element_type=jnp.float32)


def kernel(X, G_sparse, W1, b1, W2, b2):
    n, in_ch = X.shape
    n_hid = W1.shape[1]
    n_class = W2.shape[1]
    bm = _BM
    grid = (n // bm,)

    pad = (-n_class) % 128
    w2p = jnp.pad(W2, ((0, 0), (0, pad)))
    b2p = jnp.pad(b2, ((0, pad),)).reshape(1, -1)
    b1r = b1.reshape(1, -1)
    wide = n_class + pad

    params = pltpu.CompilerParams(
        dimension_semantics=("parallel",),
        vmem_limit_bytes=128 * 1024 * 1024,
    )

    a = pl.pallas_call(
        _linear_body,
        grid=grid,
        in_specs=[
            pl.BlockSpec((bm, in_ch), lambda i: (i, 0)),
            pl.BlockSpec((in_ch, n_hid), lambda i: (0, 0)),
            pl.BlockSpec((1, n_hid), lambda i: (0, 0)),
        ],
        out_specs=pl.BlockSpec((bm, n_hid), lambda i: (i, 0)),
        out_shape=jax.ShapeDtypeStruct((n, n_hid), jnp.float32),
        compiler_params=params,
    )(X, W1, b1r)

    b = pl.pallas_call(
        _layer1_body,
        grid=grid,
        in_specs=[
            pl.BlockSpec((bm, n), lambda i: (i, 0)),
            pl.BlockSpec((n, n_hid), lambda i: (0, 0)),
            pl.BlockSpec((n_hid, wide), lambda i: (0, 0)),
            pl.BlockSpec((1, wide), lambda i: (0, 0)),
        ],
        out_specs=pl.BlockSpec((bm, wide), lambda i: (i, 0)),
        out_shape=jax.ShapeDtypeStruct((n, wide), jnp.float32),
        compiler_params=params,
    )(G_sparse, a, w2p, b2p)

    out_full = pl.pallas_call(
        _layer2_body,
        grid=grid,
        in_specs=[
            pl.BlockSpec((bm, n), lambda i: (i, 0)),
            pl.BlockSpec((n, wide), lambda i: (0, 0)),
        ],
        out_specs=pl.BlockSpec((bm, wide), lambda i: (i, 0)),
        out_shape=jax.ShapeDtypeStruct((n, wide), jnp.float32),
        compiler_params=params,
    )(G_sparse, b)

    return out_full[:, :n_class]


# speedup vs baseline: 1.0107x; 1.0107x over previous
import jax
import jax.numpy as jnp
from jax.experimental import pallas as pl
from jax.experimental.pallas import tpu as pltpu

_BM = 400


def _linear_body(x_ref, w_ref, b_ref, o_ref):
    o_ref[...] = (
        jnp.dot(x_ref[...], w_ref[...], preferred_element_type=jnp.float32)
        + b_ref[...]
    )


def _layer1_body(g_ref, a_ref, w2_ref, b2_ref, o_ref):
    g = g_ref[...].astype(jnp.bfloat16)
    a = a_ref[...].astype(jnp.bfloat16)
    h = jnp.dot(g, a, preferred_element_type=jnp.float32)
    h = jnp.maximum(h, 0.0).astype(jnp.bfloat16)
    w2 = w2_ref[...].astype(jnp.bfloat16)
    o_ref[...] = (
        jnp.dot(h, w2, preferred_element_type=jnp.float32) + b2_ref[...]
    )


def _layer2_body(g_ref, b_ref, o_ref):
    g = g_ref[...].astype(jnp.bfloat16)
    b = b_ref[...].astype(jnp.bfloat16)
    o_ref[...] = jnp.dot(g, b, preferred_element_type=jnp.float32)


def kernel(X, G_sparse, W1, b1, W2, b2):
    n, in_ch = X.shape
    n_hid = W1.shape[1]
    n_class = W2.shape[1]
    bm = _BM
    grid = (n // bm,)

    pad = (-n_class) % 128
    w2p = jnp.pad(W2, ((0, 0), (0, pad)))
    b2p = jnp.pad(b2, ((0, pad),)).reshape(1, -1)
    b1r = b1.reshape(1, -1)
    wide = n_class + pad

    params = pltpu.CompilerParams(
        dimension_semantics=("parallel",),
        vmem_limit_bytes=128 * 1024 * 1024,
    )

    a = pl.pallas_call(
        _linear_body,
        grid=grid,
        in_specs=[
            pl.BlockSpec((bm, in_ch), lambda i: (i, 0)),
            pl.BlockSpec((in_ch, n_hid), lambda i: (0, 0)),
            pl.BlockSpec((1, n_hid), lambda i: (0, 0)),
        ],
        out_specs=pl.BlockSpec((bm, n_hid), lambda i: (i, 0)),
        out_shape=jax.ShapeDtypeStruct((n, n_hid), jnp.float32),
        compiler_params=params,
    )(X, W1, b1r)

    b = pl.pallas_call(
        _layer1_body,
        grid=grid,
        in_specs=[
            pl.BlockSpec((bm, n), lambda i: (i, 0)),
            pl.BlockSpec((n, n_hid), lambda i: (0, 0)),
            pl.BlockSpec((n_hid, wide), lambda i: (0, 0)),
            pl.BlockSpec((1, wide), lambda i: (0, 0)),
        ],
        out_specs=pl.BlockSpec((bm, wide), lambda i: (i, 0)),
        out_shape=jax.ShapeDtypeStruct((n, wide), jnp.float32),
        compiler_params=params,
    )(G_sparse, a, w2p, b2p)

    out_full = pl.pallas_call(
        _layer2_body,
        grid=grid,
        in_specs=[
            pl.BlockSpec((bm, n), lambda i: (i, 0)),
            pl.BlockSpec((n, wide), lambda i: (0, 0)),
        ],
        out_specs=pl.BlockSpec((bm, wide), lambda i: (i, 0)),
        out_shape=jax.ShapeDtypeStruct((n, wide), jnp.float32),
        compiler_params=params,
    )(G_sparse, b)

    return out_full[:, :n_class]
